# baseline (speedup 1.0000x reference)
import functools

import jax
import jax.numpy as jnp
from jax import lax
from jax.experimental import pallas as pl
from jax.experimental.pallas import tpu as pltpu
from jax.experimental.pallas import tpu_sc as plsc

_NF = 26
_V = 100000
_B = 16384
_ND = 13

_NC, _NS, _L = 2, 16, 16
_FPS = _NF // _NC
_BPW = _B // _NS
_G = _BPW // _L
_FA = 7
_FB = _FPS - _FA


def _sc_body(refs):
    (cat_ref, tv_ref, numt_ref, nkb_ref, flat_ref, out_ref,
     idx_a, idx_b, rows_a, rows_b, nkb_v, out_v, buf,
     sem_st, sem_fl, sem_g) = refs
    c = lax.axis_index("c")
    s = lax.axis_index("s")

    cp_nk = pltpu.async_copy(nkb_ref, nkb_v, sem_st)
    sbase = pl.multiple_of(s * _BPW, _BPW)

    def _idx_dst(j):
        return (idx_a.at[pl.ds(j * _BPW, _BPW)] if j < _FA
                else idx_b.at[pl.ds((j - _FA) * _BPW, _BPW)])

    idx_cps = [
        pltpu.async_copy(cat_ref.at[_FPS * c + j, s, 0], _idx_dst(j), sem_st)
        for j in range(_FPS)
    ]

    @pl.when(s < _FPS)
    def _():
        fld = _FPS * c + s
        pltpu.async_copy(tv_ref.at[fld, 0], buf, sem_fl).wait()
        pltpu.async_copy(buf, flat_ref.at[pl.ds(fld * _V, _V)], sem_fl).wait()

    for cp in idx_cps:
        cp.wait()
    base_off = _FPS * c * _V

    def _offs_a(g, _):
        for j in range(_FA):
            sl = pl.ds(j * _BPW + g * _L, _L)
            idx_a[sl] = idx_a[sl] + (base_off + j * _V)
        return 0
    lax.fori_loop(0, _G, _offs_a, 0)

    def _offs_b(g, _):
        for j in range(_FB):
            sl = pl.ds(j * _BPW + g * _L, _L)
            idx_b[sl] = idx_b[sl] + (base_off + (_FA + j) * _V)
        return 0
    lax.fori_loop(0, _G, _offs_b, 0)

    numt_cps = [
        pltpu.async_copy(numt_ref.at[pl.ds(d * _B + sbase, _BPW)],
                         buf.at[pl.ds(d * _BPW, _BPW)], sem_st)
        for d in range(_ND)
    ]

    plsc.subcore_barrier()

    g_a = pltpu.async_copy(flat_ref.at[idx_a], rows_a, sem_g)
    g_b = pltpu.async_copy(flat_ref.at[idx_b], rows_b, sem_g)

    cp_nk.wait()
    for cp in numt_cps:
        cp.wait()
    czero = (1 - c).astype(jnp.float32)
    nkb_vec = nkb_v[:] * jnp.broadcast_to(czero, (_L,))

    g_a.wait()

    def _gbody_a(g, _):
        sl = pl.ds(g * _L, _L)
        acc = jnp.broadcast_to(nkb_vec[_ND], (_L,))
        for d in range(_ND):
            acc = acc + nkb_vec[d] * buf[pl.ds(d * _BPW + g * _L, _L)]
        for i in range(_FA):
            acc = acc + rows_a[pl.ds(i * _BPW + g * _L, _L)]
        out_v[sl] = acc
        return 0
    lax.fori_loop(0, _G, _gbody_a, 0)

    g_b.wait()

    def _gbody_b(g, _):
        sl = pl.ds(g * _L, _L)
        acc = out_v[sl]
        for i in range(_FB):
            acc = acc + rows_b[pl.ds(i * _BPW + g * _L, _L)]
        out_v[sl] = acc
        return 0
    lax.fori_loop(0, _G, _gbody_b, 0)

    pltpu.sync_copy(out_v, out_ref.at[c, s])


_sc_linear_features = functools.partial(
    pl.kernel,
    out_type=(
        jax.ShapeDtypeStruct((_NF * _V,), jnp.float32),
        jax.ShapeDtypeStruct((_NC, _NS, _BPW), jnp.float32),
    ),
    mesh=plsc.VectorSubcoreMesh(core_axis_name="c", subcore_axis_name="s",
                                num_cores=_NC, num_subcores=_NS),
    scratch_types=[
        pltpu.VMEM((_FA * _BPW,), jnp.int32),
        pltpu.VMEM((_FB * _BPW,), jnp.int32),
        pltpu.VMEM((_FA * _BPW,), jnp.float32),
        pltpu.VMEM((_FB * _BPW,), jnp.float32),
        pltpu.VMEM((_L,), jnp.float32),
        pltpu.VMEM((_BPW,), jnp.float32),
        pltpu.VMEM((_V,), jnp.float32),
        pltpu.SemaphoreType.DMA,
        pltpu.SemaphoreType.DMA,
        pltpu.SemaphoreType.DMA,
    ],
)


def _sc_entry(*refs):
    _sc_body(refs)


_sc_call = _sc_linear_features(_sc_entry)


@jax.jit
def _run(cats, num, tables, numeric_kernel, bias):
    cats4d = jnp.concatenate(cats, axis=1).T
    cats4d = cats4d.reshape(_NF, _NS, 1, _BPW)
    numt = num.T.reshape(_ND * _B)
    nkb = jnp.concatenate(
        [numeric_kernel[:, 0], bias,
         jnp.zeros((_L - _ND - 1,), jnp.float32)])
    tv = jnp.transpose(tables, (0, 2, 1))
    flat, parts = _sc_call(cats4d, tv, numt, nkb)
    del flat
    return (parts[0] + parts[1]).reshape(_B, 1)


def kernel(cat_00, cat_01, cat_02, cat_03, cat_04, cat_05, cat_06, cat_07,
           cat_08, cat_09, cat_10, cat_11, cat_12, cat_13, cat_14, cat_15,
           cat_16, cat_17, cat_18, cat_19, cat_20, cat_21, cat_22, cat_23,
           cat_24, cat_25, num, tables, numeric_kernel, bias):
    cats = (cat_00, cat_01, cat_02, cat_03, cat_04, cat_05, cat_06, cat_07,
            cat_08, cat_09, cat_10, cat_11, cat_12, cat_13, cat_14, cat_15,
            cat_16, cat_17, cat_18, cat_19, cat_20, cat_21, cat_22, cat_23,
            cat_24, cat_25)
    return _run(cats, num, tables, numeric_kernel, bias)


# speedup vs baseline: 1.2204x; 1.2204x over previous
import functools

import jax
import jax.numpy as jnp
from jax import lax
from jax.experimental import pallas as pl
from jax.experimental.pallas import tpu as pltpu
from jax.experimental.pallas import tpu_sc as plsc

_N_FIELDS = 26
_VOCAB = 100000
_BATCH = 16384
_NUM_DIM = 13

_VPAD = 100096
_NC, _NS, _L = 2, 16, 16
_NW = _NC * _NS
_BPW = _BATCH // _NW
_G = _BPW // _L


_HF = _N_FIELDS // 2
_HW = _HF * _BPW


def _sc_body(table_ref, idx_ref, numt_ref, nkb_ref, out_ref,
             idx_a, idx_b, rows_a, rows_b, numt_v, nkb_v, out_v,
             sem_st, sem_a, sem_b):
    c = lax.axis_index("c")
    s = lax.axis_index("s")
    wid = s * _NC + c
    base = wid * _BPW

    idx_cps = [
        pltpu.async_copy(
            idx_ref.at[j, wid, 0],
            (idx_a.at[pl.ds(j * _BPW, _BPW)] if j < _HF
             else idx_b.at[pl.ds((j - _HF) * _BPW, _BPW)]),
            sem_st)
        for j in range(_N_FIELDS)
    ]
    cp_nt = pltpu.async_copy(numt_ref.at[:, pl.ds(base, _BPW)], numt_v,
                             sem_st)
    cp_nk = pltpu.async_copy(nkb_ref, nkb_v, sem_st)

    for cp in idx_cps:
        cp.wait()
    g_a = pltpu.async_copy(table_ref.at[idx_a], rows_a, sem_a)
    g_b = pltpu.async_copy(table_ref.at[idx_b], rows_b, sem_b)

    cp_nt.wait()
    cp_nk.wait()
    nkb_vec = nkb_v[:]

    g_a.wait()

    def _gbody_a(g, _):
        sl = pl.ds(g * _L, _L)
        acc = jnp.broadcast_to(nkb_vec[_NUM_DIM], (_L,))
        for d in range(_NUM_DIM):
            acc = acc + nkb_vec[d] * numt_v[d, sl]
        for i in range(_HF):
            acc = acc + rows_a[pl.ds(i * _BPW + g * _L, _L)]
        out_v[sl] = acc
        return 0
    lax.fori_loop(0, _G, _gbody_a, 0)

    g_b.wait()

    def _gbody_b(g, _):
        sl = pl.ds(g * _L, _L)
        acc = out_v[sl]
        for i in range(_HF):
            acc = acc + rows_b[pl.ds(i * _BPW + g * _L, _L)]
        out_v[sl] = acc
        return 0
    lax.fori_loop(0, _G, _gbody_b, 0)

    obase = pl.multiple_of(wid * _BPW, _BPW)
    pltpu.sync_copy(out_v, out_ref.at[pl.ds(obase, _BPW)])


@functools.partial(
    pl.kernel,
    out_type=jax.ShapeDtypeStruct((_N_FIELDS * _VOCAB,), jnp.float32),
    mesh=plsc.VectorSubcoreMesh(core_axis_name="c", subcore_axis_name="s",
                                num_cores=_NC, num_subcores=_NS),
    scratch_types=[
        pltpu.VMEM((_VOCAB,), jnp.float32),
    ],
)
def _sc_flatten_tables(tv_ref, flat_ref, buf_v):
    c = lax.axis_index("c")
    s = lax.axis_index("s")
    wid = s * _NC + c

    @pl.when(wid < _N_FIELDS)
    def _():
        pltpu.sync_copy(tv_ref.at[wid, 0], buf_v)
        pltpu.sync_copy(buf_v, flat_ref.at[pl.ds(wid * _VOCAB, _VOCAB)])


@functools.partial(
    pl.kernel,
    out_type=jax.ShapeDtypeStruct((_BATCH,), jnp.float32),
    mesh=plsc.VectorSubcoreMesh(core_axis_name="c", subcore_axis_name="s",
                                num_cores=_NC, num_subcores=_NS),
    scratch_types=[
        pltpu.VMEM((_HW,), jnp.int32),
        pltpu.VMEM((_HW,), jnp.int32),
        pltpu.VMEM((_HW,), jnp.float32),
        pltpu.VMEM((_HW,), jnp.float32),
        pltpu.VMEM((_NUM_DIM, _BPW), jnp.float32),
        pltpu.VMEM((_L,), jnp.float32),
        pltpu.VMEM((_BPW,), jnp.float32),
        pltpu.SemaphoreType.DMA,
        pltpu.SemaphoreType.DMA,
        pltpu.SemaphoreType.DMA,
    ],
)
def _sc_linear_features(*args):
    _sc_body(*args)


@jax.jit
def _run(cats, num, tables, numeric_kernel, bias):
    off = (jnp.arange(_N_FIELDS, dtype=jnp.int32) * _VOCAB)[:, None]
    idx = jnp.concatenate(cats, axis=1).T + off
    idx = idx.reshape(_N_FIELDS, _NW, 1, _BPW)
    numt = num.T
    nkb = jnp.concatenate(
        [numeric_kernel[:, 0], bias,
         jnp.zeros((_L - _NUM_DIM - 1,), jnp.float32)])
    table_flat = _sc_flatten_tables(jnp.transpose(tables, (0, 2, 1)))
    out = _sc_linear_features(table_flat, idx, numt, nkb)
    return out.reshape(_BATCH, 1)


def kernel(cat_00, cat_01, cat_02, cat_03, cat_04, cat_05, cat_06, cat_07,
           cat_08, cat_09, cat_10, cat_11, cat_12, cat_13, cat_14, cat_15,
           cat_16, cat_17, cat_18, cat_19, cat_20, cat_21, cat_22, cat_23,
           cat_24, cat_25, num, tables, numeric_kernel, bias):
    cats = (cat_00, cat_01, cat_02, cat_03, cat_04, cat_05, cat_06, cat_07,
            cat_08, cat_09, cat_10, cat_11, cat_12, cat_13, cat_14, cat_15,
            cat_16, cat_17, cat_18, cat_19, cat_20, cat_21, cat_22, cat_23,
            cat_24, cat_25)
    return _run(cats, num, tables, numeric_kernel, bias)


# speedup vs baseline: 1.2248x; 1.0036x over previous
import functools

import jax
import jax.numpy as jnp
from jax import lax
from jax.experimental import pallas as pl
from jax.experimental.pallas import tpu as pltpu
from jax.experimental.pallas import tpu_sc as plsc

_N_FIELDS = 26
_VOCAB = 100000
_BATCH = 16384
_NUM_DIM = 13

_VPAD = 100096
_NC, _NS, _L = 2, 16, 16
_NW = _NC * _NS
_BPW = _BATCH // _NW
_G = _BPW // _L


_CHUNKS = (7, 7, 6, 6)
_CSTART = (0, 7, 14, 20)


def _sc_body(table_ref, idx_ref, numt_ref, nkb_ref, out_ref,
             idx_c, rows_c, numt_v, nkb_v, out_v, sem_st, sems):
    c = lax.axis_index("c")
    s = lax.axis_index("s")
    wid = s * _NC + c
    base = wid * _BPW

    idx_cps = [
        [pltpu.async_copy(
            idx_ref.at[_CSTART[k] + j, wid, 0],
            idx_c[k].at[pl.ds(j * _BPW, _BPW)], sem_st)
         for j in range(_CHUNKS[k])]
        for k in range(4)
    ]
    cp_nt = pltpu.async_copy(numt_ref.at[:, pl.ds(base, _BPW)], numt_v,
                             sem_st)
    cp_nk = pltpu.async_copy(nkb_ref, nkb_v, sem_st)

    gs = []
    for k in range(4):
        for cp in idx_cps[k]:
            cp.wait()
        gs.append(pltpu.async_copy(table_ref.at[idx_c[k]], rows_c[k],
                                   sems[k]))

    cp_nt.wait()
    cp_nk.wait()
    nkb_vec = nkb_v[:]

    gs[0].wait()

    def _gbody0(g, _):
        sl = pl.ds(g * _L, _L)
        acc = jnp.broadcast_to(nkb_vec[_NUM_DIM], (_L,))
        for d in range(_NUM_DIM):
            acc = acc + nkb_vec[d] * numt_v[d, sl]
        for i in range(_CHUNKS[0]):
            acc = acc + rows_c[0][pl.ds(i * _BPW + g * _L, _L)]
        out_v[sl] = acc
        return 0
    lax.fori_loop(0, _G, _gbody0, 0)

    for k in range(1, 4):
        gs[k].wait()

        def _gbodyk(g, _, k=k):
            sl = pl.ds(g * _L, _L)
            acc = out_v[sl]
            for i in range(_CHUNKS[k]):
                acc = acc + rows_c[k][pl.ds(i * _BPW + g * _L, _L)]
            out_v[sl] = acc
            return 0
        lax.fori_loop(0, _G, _gbodyk, 0)

    obase = pl.multiple_of(wid * _BPW, _BPW)
    pltpu.sync_copy(out_v, out_ref.at[pl.ds(obase, _BPW)])


@functools.partial(
    pl.kernel,
    out_type=jax.ShapeDtypeStruct((_N_FIELDS * _VOCAB,), jnp.float32),
    mesh=plsc.VectorSubcoreMesh(core_axis_name="c", subcore_axis_name="s",
                                num_cores=_NC, num_subcores=_NS),
    scratch_types=[
        pltpu.VMEM((_VOCAB,), jnp.float32),
    ],
)
def _sc_flatten_tables(tv_ref, flat_ref, buf_v):
    c = lax.axis_index("c")
    s = lax.axis_index("s")
    wid = s * _NC + c

    @pl.when(wid < _N_FIELDS)
    def _():
        pltpu.sync_copy(tv_ref.at[wid, 0], buf_v)
        pltpu.sync_copy(buf_v, flat_ref.at[pl.ds(wid * _VOCAB, _VOCAB)])


@functools.partial(
    pl.kernel,
    out_type=jax.ShapeDtypeStruct((_BATCH,), jnp.float32),
    mesh=plsc.VectorSubcoreMesh(core_axis_name="c", subcore_axis_name="s",
                                num_cores=_NC, num_subcores=_NS),
    scratch_types=(
        [pltpu.VMEM((_CHUNKS[k] * _BPW,), jnp.int32) for k in range(4)]
        + [pltpu.VMEM((_CHUNKS[k] * _BPW,), jnp.float32) for k in range(4)]
        + [
            pltpu.VMEM((_NUM_DIM, _BPW), jnp.float32),
            pltpu.VMEM((_L,), jnp.float32),
            pltpu.VMEM((_BPW,), jnp.float32),
            pltpu.SemaphoreType.DMA,
            pltpu.SemaphoreType.DMA,
            pltpu.SemaphoreType.DMA,
            pltpu.SemaphoreType.DMA,
            pltpu.SemaphoreType.DMA,
        ]
    ),
)
def _sc_linear_features(table_ref, idx_ref, numt_ref, nkb_ref, out_ref, *scr):
    idx_c = scr[:4]
    rows_c = scr[4:8]
    numt_v, nkb_v, out_v, sem_st = scr[8:12]
    sems = scr[12:16]
    _sc_body(table_ref, idx_ref, numt_ref, nkb_ref, out_ref,
             idx_c, rows_c, numt_v, nkb_v, out_v, sem_st, sems)


@jax.jit
def _run(cats, num, tables, numeric_kernel, bias):
    off = (jnp.arange(_N_FIELDS, dtype=jnp.int32) * _VOCAB)[:, None]
    idx = jnp.concatenate(cats, axis=1).T + off
    idx = idx.reshape(_N_FIELDS, _NW, 1, _BPW)
    numt = num.T
    nkb = jnp.concatenate(
        [numeric_kernel[:, 0], bias,
         jnp.zeros((_L - _NUM_DIM - 1,), jnp.float32)])
    table_flat = _sc_flatten_tables(jnp.transpose(tables, (0, 2, 1)))
    out = _sc_linear_features(table_flat, idx, numt, nkb)
    return out.reshape(_BATCH, 1)


def kernel(cat_00, cat_01, cat_02, cat_03, cat_04, cat_05, cat_06, cat_07,
           cat_08, cat_09, cat_10, cat_11, cat_12, cat_13, cat_14, cat_15,
           cat_16, cat_17, cat_18, cat_19, cat_20, cat_21, cat_22, cat_23,
           cat_24, cat_25, num, tables, numeric_kernel, bias):
    cats = (cat_00, cat_01, cat_02, cat_03, cat_04, cat_05, cat_06, cat_07,
            cat_08, cat_09, cat_10, cat_11, cat_12, cat_13, cat_14, cat_15,
            cat_16, cat_17, cat_18, cat_19, cat_20, cat_21, cat_22, cat_23,
            cat_24, cat_25)
    return _run(cats, num, tables, numeric_kernel, bias)
